# Initial kernel scaffold; baseline (speedup 1.0000x reference)
import jax
import jax.numpy as jnp
from jax.experimental import pallas as pl


def kernel(x, edge_index, W1, b1, Wmu, bmu, Wls, bls):
    raise NotImplementedError("write your pallas kernel here")


# speedup vs baseline: 8.7696x; 8.7696x over previous
import functools

import jax
import jax.numpy as jnp
from jax import lax
from jax.experimental import pallas as pl
from jax.experimental.pallas import tpu as pltpu
from jax.experimental.pallas import tpu_sc as plsc

N = 10000
E = 320000
D = 128
D_OUT = 64

CHUNK = 128
NW = 32
K_PER_W = 80
NCH = K_PER_W * NW
PAD_E = NCH * CHUNK
RPT = 632
NP = RPT * 16

_mesh = plsc.VectorSubcoreMesh(core_axis_name="c", subcore_axis_name="s")


@functools.partial(
    pl.kernel,
    out_type=jax.ShapeDtypeStruct((2, NP, D), jnp.float32),
    mesh=_mesh,
    scratch_types=[
        pltpu.VMEM((K_PER_W, CHUNK), jnp.int32),
        pltpu.VMEM((CHUNK, D), jnp.float32),
        pltpu.VMEM_SHARED((NP, D), jnp.float32),
    ],
)
def _deg_sc(dst_hbm, ones_hbm, zeros_hbm, out_hbm, didx_v, ones_v, acc):
    c = lax.axis_index("c")
    s = lax.axis_index("s")
    wid = s * 2 + c

    pltpu.sync_copy(dst_hbm.at[pl.ds(wid * K_PER_W, K_PER_W)], didx_v)
    pltpu.sync_copy(ones_hbm, ones_v)
    pltpu.sync_copy(zeros_hbm, acc.at[pl.ds(s * RPT, RPT)])
    plsc.subcore_barrier()

    def body(j, carry):
        pltpu.sync_copy(ones_v, acc.at[didx_v.at[j]], add=True)
        return carry

    lax.fori_loop(0, K_PER_W, body, 0)
    plsc.subcore_barrier()
    pltpu.sync_copy(acc.at[pl.ds(s * RPT, RPT)],
                    out_hbm.at[c, pl.ds(s * RPT, RPT)])


@functools.partial(
    pl.kernel,
    out_type=jax.ShapeDtypeStruct((2, NP, D), jnp.float32),
    mesh=_mesh,
    scratch_types=[
        pltpu.VMEM((K_PER_W, CHUNK), jnp.int32),
        pltpu.VMEM((K_PER_W, CHUNK), jnp.int32),
        pltpu.VMEM((CHUNK, D), jnp.float32),
        pltpu.VMEM_SHARED((NP, D), jnp.float32),
    ],
)
def _agg_sc(feat_hbm, src_hbm, dst_hbm, zeros_hbm, out_hbm, sidx_v, didx_v, rows_v, acc):
    c = lax.axis_index("c")
    s = lax.axis_index("s")
    wid = s * 2 + c

    pltpu.sync_copy(src_hbm.at[pl.ds(wid * K_PER_W, K_PER_W)], sidx_v)
    pltpu.sync_copy(dst_hbm.at[pl.ds(wid * K_PER_W, K_PER_W)], didx_v)
    pltpu.sync_copy(zeros_hbm, acc.at[pl.ds(s * RPT, RPT)])
    plsc.subcore_barrier()

    def body(j, carry):
        pltpu.sync_copy(feat_hbm.at[sidx_v.at[j]], rows_v)
        pltpu.sync_copy(rows_v, acc.at[didx_v.at[j]], add=True)
        return carry

    lax.fori_loop(0, K_PER_W, body, 0)
    plsc.subcore_barrier()
    pltpu.sync_copy(acc.at[pl.ds(s * RPT, RPT)],
                    out_hbm.at[c, pl.ds(s * RPT, RPT)])


_BN = 1000


def _xs_body(x_ref, degp_ref, xs_ref):
    deg = degp_ref[0, :, 0:1] + degp_ref[1, :, 0:1] + 1.0
    dinv = lax.rsqrt(deg)
    xs_ref[...] = x_ref[...] * dinv


_xs_tc = pl.pallas_call(
    _xs_body,
    grid=(N // _BN,),
    in_specs=[
        pl.BlockSpec((_BN, D), lambda i: (i, 0)),
        pl.BlockSpec((2, _BN, D), lambda i: (0, i, 0)),
    ],
    out_specs=pl.BlockSpec((_BN, D), lambda i: (i, 0)),
    out_shape=jax.ShapeDtypeStruct((NP, D), jnp.float32),
)


def _l1_body(p_ref, xs_ref, degp_ref, w_ref, b_ref, hs_ref):
    deg = degp_ref[0, :, 0:1] + degp_ref[1, :, 0:1] + 1.0
    dinv = lax.rsqrt(deg)
    ax = (p_ref[0] + p_ref[1] + xs_ref[...]) * dinv
    h = jnp.dot(ax, w_ref[...], preferred_element_type=jnp.float32,
                precision=lax.Precision.HIGHEST)
    h = jnp.maximum(h + b_ref[...], 0.0)
    hs_ref[...] = h * dinv


_l1_tc = pl.pallas_call(
    _l1_body,
    grid=(N // _BN,),
    in_specs=[
        pl.BlockSpec((2, _BN, D), lambda i: (0, i, 0)),
        pl.BlockSpec((_BN, D), lambda i: (i, 0)),
        pl.BlockSpec((2, _BN, D), lambda i: (0, i, 0)),
        pl.BlockSpec((D, D), lambda i: (0, 0)),
        pl.BlockSpec((1, D), lambda i: (0, 0)),
    ],
    out_specs=pl.BlockSpec((_BN, D), lambda i: (i, 0)),
    out_shape=jax.ShapeDtypeStruct((NP, D), jnp.float32),
)


def _l2_body(q_ref, hs_ref, degp_ref, w_ref, b_ref, o_ref):
    deg = degp_ref[0, :, 0:1] + degp_ref[1, :, 0:1] + 1.0
    dinv = lax.rsqrt(deg)
    ah = (q_ref[0] + q_ref[1] + hs_ref[...]) * dinv
    o_ref[...] = jnp.dot(ah, w_ref[...], preferred_element_type=jnp.float32,
                         precision=lax.Precision.HIGHEST) + b_ref[...]


_l2_tc = pl.pallas_call(
    _l2_body,
    grid=(N // _BN,),
    in_specs=[
        pl.BlockSpec((2, _BN, D), lambda i: (0, i, 0)),
        pl.BlockSpec((_BN, D), lambda i: (i, 0)),
        pl.BlockSpec((2, _BN, D), lambda i: (0, i, 0)),
        pl.BlockSpec((D, D), lambda i: (0, 0)),
        pl.BlockSpec((1, D), lambda i: (0, 0)),
    ],
    out_specs=pl.BlockSpec((_BN, D), lambda i: (i, 0)),
    out_shape=jax.ShapeDtypeStruct((N, D), jnp.float32),
)


def kernel(x, edge_index, W1, b1, Wmu, bmu, Wls, bls):
    src = edge_index[0]
    dst = edge_index[1]
    pad = jnp.full((PAD_E - E,), N, dtype=jnp.int32)
    srcp = jnp.concatenate([src, pad]).reshape(NCH, CHUNK)
    dstp = jnp.concatenate([dst, pad]).reshape(NCH, CHUNK)

    ones128 = jnp.ones((CHUNK, D), jnp.float32)
    zeros128 = jnp.zeros((RPT, D), jnp.float32)

    degp = _deg_sc(dstp, ones128, zeros128)

    xs = _xs_tc(x, degp)
    p = _agg_sc(xs, srcp, dstp, zeros128)
    hs = _l1_tc(p, xs, degp, W1, b1.reshape(1, D))

    q = _agg_sc(hs, srcp, dstp, zeros128)
    W2 = jnp.concatenate([Wmu, Wls], axis=1)
    b2 = jnp.concatenate([bmu, bls]).reshape(1, D)
    out = _l2_tc(q, hs, degp, W2, b2)
    return out[:, :D_OUT], out[:, D_OUT:]
